# baseline (speedup 1.0000x reference)
import functools

import jax
import jax.numpy as jnp
from jax import lax
from jax.experimental import pallas as pl
from jax.experimental.pallas import tpu as pltpu
from jax.experimental.pallas import tpu_sc as plsc

D = 32
NC = 2
NS = 16
L = 16
NW = NC * NS


def _make_kernel(B):
    BPW = B // NW
    G = BPW // L
    mesh = plsc.VectorSubcoreMesh(core_axis_name="c", subcore_axis_name="s")

    @functools.partial(
        pl.kernel,
        mesh=mesh,
        out_type=jax.ShapeDtypeStruct((B,), jnp.float32),
        compiler_params=pltpu.CompilerParams(
            use_tc_tiling_on_sc=False, needs_layout_passes=False
        ),
        scratch_types=[
            pltpu.VMEM((BPW,), jnp.int32),
            pltpu.VMEM((BPW,), jnp.int32),
            pltpu.VMEM((BPW, D), jnp.float32),
            pltpu.VMEM((BPW, D), jnp.float32),
            pltpu.VMEM((BPW,), jnp.float32),
            pltpu.SemaphoreType.DMA,
            pltpu.SemaphoreType.DMA,
        ],
    )
    def mf_kernel(aid1_hbm, aid2_hbm, table_hbm, out_hbm,
                  idx1_v, idx2_v, rows1_v, rows2_v, out_v, sem1, sem2):
        wid = lax.axis_index("s") * NC + lax.axis_index("c")
        base = wid * BPW
        pltpu.sync_copy(aid1_hbm.at[pl.ds(base, BPW)], idx1_v)
        pltpu.sync_copy(aid2_hbm.at[pl.ds(base, BPW)], idx2_v)
        cp1 = pltpu.async_copy(table_hbm.at[idx1_v], rows1_v, sem1)
        cp2 = pltpu.async_copy(table_hbm.at[idx2_v], rows2_v, sem2)
        cp1.wait()
        cp2.wait()

        def body(g, carry):
            row = g * L + lax.iota(jnp.int32, L)
            acc = jnp.zeros((L,), jnp.float32)
            for d in range(D):
                col = jnp.full((L,), d, jnp.int32)
                a = plsc.load_gather(rows1_v, [row, col])
                b = plsc.load_gather(rows2_v, [row, col])
                acc = acc + a * b
            out_v[pl.ds(g * L, L)] = acc
            return carry

        lax.fori_loop(0, G, body, 0)
        pltpu.sync_copy(out_v, out_hbm.at[pl.ds(base, BPW)])

    return mf_kernel


def kernel(aid1, aid2, table):
    return _make_kernel(aid1.shape[0])(aid1, aid2, table)
